# baseline (speedup 1.0000x reference)
import functools

import jax
import jax.numpy as jnp
from jax import lax
from jax.experimental import pallas as pl
from jax.experimental.pallas import tpu as pltpu
from jax.experimental.pallas import tpu_sc as plsc

BATCH = 16384
FEAT = 64
NUM_CLASSES = 100000

NC = 2
NS = 16
NW = NC * NS
BPW = BATCH // NW
TBL = 100096
TBL_PER_TILE = TBL // NS


def _sc_body(labels_hbm, feature_hbm, centers_hbm, sumsq_hbm, num_hbm,
             idx_v, cidx_v, rows_v, feat_v, num_v, sumsq_v, zeros_v, ones_v,
             table, rows_sem, feat_sem):
    c = lax.axis_index("c")
    s = lax.axis_index("s")
    w = c * NS + s

    pltpu.sync_copy(labels_hbm.at[pl.ds(w * 4, 4)], idx_v)

    row_cps = [
        pltpu.async_copy(centers_hbm.at[idx_v.at[j]],
                         rows_v.at[pl.ds(j * 128, 128)], rows_sem)
        for j in range(4)
    ]
    feat_cp = pltpu.async_copy(feature_hbm.at[pl.ds(w * BPW, BPW)],
                               feat_v, feat_sem)

    def _zbody(k, _):
        zeros_v[pl.ds(k * 16, 16)] = jnp.zeros((16,), jnp.float32)
        return ()
    lax.fori_loop(0, TBL_PER_TILE // 16, _zbody, ())
    for k in range(8):
        ones_v[pl.ds(k * 16, 16)] = jnp.ones((16,), jnp.float32)

    pltpu.sync_copy(zeros_v, table.at[pl.ds(s * TBL_PER_TILE, TBL_PER_TILE)])
    plsc.subcore_barrier()
    pltpu.sync_copy(labels_hbm.at[pl.ds(s * 8, 8)], cidx_v)
    for j in range(8):
        pltpu.sync_copy(ones_v, table.at[cidx_v.at[j]], add=True)
    plsc.subcore_barrier()

    for j in range(4):
        pltpu.sync_copy(table.at[idx_v.at[j]], num_v.at[pl.ds(j * 128, 128)])

    for cp in row_cps:
        cp.wait()
    feat_cp.wait()

    lane = lax.iota(jnp.int32, 16)

    def _gbody(g, _):
        acc = jnp.zeros((16,), jnp.float32)
        for j in range(16):
            i = g * 16 + j
            t = jnp.zeros((16,), jnp.float32)
            for ch in range(4):
                d = (feat_v[i, pl.ds(ch * 16, 16)]
                     - rows_v[i, pl.ds(ch * 16, 16)])
                t = t + d * d
            acc = jnp.where(lane == j, jnp.sum(t), acc)
        sumsq_v[pl.ds(g * 16, 16)] = acc
        return ()
    lax.fori_loop(0, BPW // 16, _gbody, ())

    pltpu.sync_copy(sumsq_v, sumsq_hbm.at[pl.ds(w * BPW, BPW)])
    pltpu.sync_copy(num_v, num_hbm.at[pl.ds(w * BPW, BPW)])


@jax.jit
def _sc_stage(labels2d, feature, centers):
    mesh = plsc.VectorSubcoreMesh(core_axis_name="c", subcore_axis_name="s")
    fn = pl.kernel(
        _sc_body,
        out_type=(
            jax.ShapeDtypeStruct((BATCH,), jnp.float32),
            jax.ShapeDtypeStruct((BATCH,), jnp.float32),
        ),
        mesh=mesh,
        compiler_params=pltpu.CompilerParams(
            needs_layout_passes=False, use_tc_tiling_on_sc=False),
        scratch_types=[
            pltpu.VMEM((4, 128), jnp.int32),
            pltpu.VMEM((8, 128), jnp.int32),
            pltpu.VMEM((BPW, FEAT), jnp.float32),
            pltpu.VMEM((BPW, FEAT), jnp.float32),
            pltpu.VMEM((BPW,), jnp.float32),
            pltpu.VMEM((BPW,), jnp.float32),
            pltpu.VMEM((TBL_PER_TILE,), jnp.float32),
            pltpu.VMEM((128,), jnp.float32),
            pltpu.VMEM_SHARED((TBL,), jnp.float32),
            pltpu.SemaphoreType.DMA,
            pltpu.SemaphoreType.DMA,
        ],
    )
    return fn(labels2d, feature, centers)


def _loss_body(sumsq_ref, num_ref, out_ref):
    dist = jnp.sqrt(sumsq_ref[...])
    loss = jnp.sum(dist / num_ref[...]) * (1.0 / BATCH)
    out_ref[...] = loss.reshape(1, 1)


@jax.jit
def _tc_stage(sumsq, num):
    out = pl.pallas_call(
        _loss_body,
        out_shape=jax.ShapeDtypeStruct((1, 1), jnp.float32),
    )(sumsq.reshape(128, 128), num.reshape(128, 128))
    return out[0, 0]


def kernel(feature, label, centers):
    labels2d = jnp.asarray(label, jnp.int32).reshape(128, 128)
    sumsq, num = _sc_stage(labels2d, feature, centers)
    return _tc_stage(sumsq, num)
